# Initial kernel scaffold; baseline (speedup 1.0000x reference)
import jax
import jax.numpy as jnp
from jax.experimental import pallas as pl


def kernel(x, edge_index, W1, b1, W2, b2):
    raise NotImplementedError("write your pallas kernel here")


# speedup vs baseline: 11.2880x; 11.2880x over previous
import functools

import jax
import jax.numpy as jnp
from jax import lax
from jax.experimental import pallas as pl
from jax.experimental.pallas import tpu as pltpu
from jax.experimental.pallas import tpu_sc as plsc

N = 10000
E = 320000
D_IN = 128
D_HID = 128
D_OUT = 64

NC = 2
NS = 16
NW = NC * NS

N_PAD = 10240
EB = 128
NB = (E // NW + EB - 1) // EB
NB = 80
EPT = NB * EB
E_PAD = NW * EPT
STRIPE = N_PAD // NS


def _make_agg(D):
  mesh = plsc.VectorSubcoreMesh(core_axis_name="c", subcore_axis_name="s")

  @functools.partial(
      pl.kernel,
      out_type=jax.ShapeDtypeStruct((NC, N_PAD, D), jnp.float32),
      mesh=mesh,
      scratch_types=[
          pltpu.VMEM((NB, EB), jnp.int32),
          pltpu.VMEM((NB, EB), jnp.int32),
          pltpu.VMEM((EB, D), jnp.float32),
          pltpu.VMEM_SHARED((N_PAD, D), jnp.float32),
          pltpu.SemaphoreType.DMA,
      ],
      compiler_params=pltpu.CompilerParams(use_tc_tiling_on_sc=False),
  )
  def agg(h_hbm, src_hbm, dst_hbm, out_hbm, src_v, dst_v, rows_v,
          acc_sh, sem):
    cid = lax.axis_index("c")
    sid = lax.axis_index("s")
    wid = cid * NS + sid

    pltpu.sync_copy(src_hbm.at[wid], src_v)
    pltpu.sync_copy(dst_hbm.at[wid], dst_v)

    z16 = jnp.zeros((16,), jnp.float32)

    @pl.loop(0, EB)
    def _(r):
      for k in range(D // 16):
        rows_v[r, pl.ds(k * 16, 16)] = z16

    base = sid * STRIPE
    for kk in range(STRIPE // EB):
      pltpu.sync_copy(rows_v, acc_sh.at[pl.ds(base + kk * EB, EB)])
    plsc.subcore_barrier()

    @pl.loop(0, NB)
    def _(j):
      pltpu.async_copy(h_hbm.at[src_v.at[j]], rows_v, sem).wait()
      pltpu.sync_copy(rows_v, acc_sh.at[dst_v.at[j]], add=True)

    plsc.subcore_barrier()
    pltpu.sync_copy(acc_sh.at[pl.ds(base, STRIPE)],
                    out_hbm.at[cid, pl.ds(base, STRIPE)])

  return agg


_agg128 = _make_agg(D_HID)
_agg64 = _make_agg(D_OUT)

_deg_mesh = plsc.VectorSubcoreMesh(core_axis_name="c", subcore_axis_name="s")


@functools.partial(
    pl.kernel,
    out_type=jax.ShapeDtypeStruct((NC, N_PAD), jnp.float32),
    mesh=_deg_mesh,
    scratch_types=[
        pltpu.VMEM((NB, EB), jnp.int32),
        pltpu.VMEM((EB,), jnp.float32),
        pltpu.VMEM((EB,), jnp.float32),
        pltpu.VMEM_SHARED((N_PAD,), jnp.float32),
        pltpu.SemaphoreType.DMA,
    ],
)
def _deg(dst_hbm, out_hbm, dst_v, ones_v, zero_v, acc_sh, sem):
  del sem
  cid = lax.axis_index("c")
  sid = lax.axis_index("s")
  wid = cid * NS + sid

  pltpu.sync_copy(dst_hbm.at[wid], dst_v)

  z16 = jnp.zeros((16,), jnp.float32)
  o16 = jnp.ones((16,), jnp.float32)
  for k in range(EB // 16):
    zero_v[pl.ds(k * 16, 16)] = z16
    ones_v[pl.ds(k * 16, 16)] = o16

  base = sid * STRIPE
  for kk in range(STRIPE // EB):
    pltpu.sync_copy(zero_v, acc_sh.at[pl.ds(base + kk * EB, EB)])
  plsc.subcore_barrier()

  @pl.loop(0, NB)
  def _(j):
    pltpu.sync_copy(ones_v, acc_sh.at[dst_v.at[j]], add=True)

  plsc.subcore_barrier()
  pltpu.sync_copy(acc_sh.at[pl.ds(base, STRIPE)],
                  out_hbm.at[cid, pl.ds(base, STRIPE)])


def _tc_a(deg_p, x_pad, W1):
  def body(deg_ref, x_ref, w_ref, h_ref):
    dis = lax.rsqrt(deg_ref[0] + deg_ref[1] + 1.0)[:, None]
    h = jnp.dot(x_ref[...], w_ref[...], preferred_element_type=jnp.float32)
    h_ref[...] = h * dis

  return pl.pallas_call(
      body,
      out_shape=jax.ShapeDtypeStruct((N_PAD, D_HID), jnp.float32),
  )(deg_p, x_pad, W1)


def _tc_b(deg_p, agg1, h1, b1, W2):
  def body(deg_ref, a_ref, h_ref, b_ref, w_ref, o_ref):
    dis = lax.rsqrt(deg_ref[0] + deg_ref[1] + 1.0)[:, None]
    t = (a_ref[0] + a_ref[1] + h_ref[...]) * dis + b_ref[...]
    t = jnp.maximum(t, 0.0)
    o_ref[...] = jnp.dot(
        t, w_ref[...], preferred_element_type=jnp.float32) * dis

  return pl.pallas_call(
      body,
      out_shape=jax.ShapeDtypeStruct((N_PAD, D_OUT), jnp.float32),
  )(deg_p, agg1, h1, b1, W2)


def _tc_c(deg_p, agg2, h2, b2):
  def body(deg_ref, a_ref, h_ref, b_ref, o_ref):
    dis = lax.rsqrt(deg_ref[0] + deg_ref[1] + 1.0)[:, None]
    o_ref[...] = (a_ref[0] + a_ref[1] + h_ref[...]) * dis + b_ref[...]

  return pl.pallas_call(
      body,
      out_shape=jax.ShapeDtypeStruct((N_PAD, D_OUT), jnp.float32),
  )(deg_p, agg2, h2, b2)


def kernel(x, edge_index, W1, b1, W2, b2):
  src = edge_index[0]
  dst = edge_index[1]
  pad_e = E_PAD - E
  srcp = jnp.concatenate(
      [src, jnp.zeros((pad_e,), jnp.int32)]).reshape(NW, NB, EB)
  dstp = jnp.concatenate(
      [dst, jnp.full((pad_e,), N, jnp.int32)]).reshape(NW, NB, EB)
  xp = jnp.pad(x, ((0, N_PAD - N), (0, 0)))

  deg_p = _deg(dstp)
  h1 = _tc_a(deg_p, xp, W1)
  agg1 = _agg128(h1, srcp, dstp)
  h2 = _tc_b(deg_p, agg1, h1, b1, W2)
  agg2 = _agg64(h2, srcp, dstp)
  out = _tc_c(deg_p, agg2, h2, b2)
  return out[:N]


# speedup vs baseline: 12.2635x; 1.0864x over previous
import functools

import jax
import jax.numpy as jnp
from jax import lax
from jax.experimental import pallas as pl
from jax.experimental.pallas import tpu as pltpu
from jax.experimental.pallas import tpu_sc as plsc

N = 10000
E = 320000
D_IN = 128
D_HID = 128
D_OUT = 64

NC = 2
NS = 16
NW = NC * NS

N_PAD = 10240
EB = 128
NB = (E // NW + EB - 1) // EB
NB = 80
EPT = NB * EB
E_PAD = NW * EPT
STRIPE = N_PAD // NS


def _make_agg(D):
  mesh = plsc.VectorSubcoreMesh(core_axis_name="c", subcore_axis_name="s")

  CH = 16
  NCH = NB // CH

  @functools.partial(
      pl.kernel,
      out_type=jax.ShapeDtypeStruct((NC, N_PAD, D), jnp.float32),
      mesh=mesh,
      scratch_types=[
          pltpu.VMEM((2, CH, EB), jnp.int32),
          pltpu.VMEM((2, CH, EB), jnp.int32),
          pltpu.VMEM((2, EB, D), jnp.float32),
          pltpu.VMEM_SHARED((N_PAD, D), jnp.float32),
          pltpu.SemaphoreType.DMA,
          pltpu.SemaphoreType.DMA,
          pltpu.SemaphoreType.DMA,
      ],
      compiler_params=pltpu.CompilerParams(use_tc_tiling_on_sc=False),
  )
  def agg(h_hbm, src_hbm, dst_hbm, out_hbm, src_v, dst_v, rows_v,
          acc_sh, gsem, ssem, isem):
    cid = lax.axis_index("c")
    sid = lax.axis_index("s")
    wid = cid * NS + sid

    z16 = jnp.zeros((16,), jnp.float32)

    @pl.loop(0, EB)
    def _(r):
      for k in range(D // 16):
        rows_v[0, r, pl.ds(k * 16, 16)] = z16

    base = sid * STRIPE
    for kk in range(STRIPE // EB):
      pltpu.sync_copy(rows_v.at[0], acc_sh.at[pl.ds(base + kk * EB, EB)])
    plsc.subcore_barrier()

    pltpu.sync_copy(src_hbm.at[wid, pl.ds(0, CH)], src_v.at[0])
    pltpu.sync_copy(dst_hbm.at[wid, pl.ds(0, CH)], dst_v.at[0])
    pltpu.async_copy(h_hbm.at[src_v.at[0, 0]], rows_v.at[0], gsem)

    @pl.loop(0, NCH)
    def _(c):
      cpar = c % 2

      @pl.loop(0, CH)
      def _(j):
        jj = c * CH + j
        par = jj % 2
        pltpu.make_async_copy(h_hbm.at[src_v.at[0, 0]], rows_v.at[par],
                              gsem).wait()
        @pl.when(jj >= 1)
        def _():
          pltpu.make_async_copy(rows_v.at[1 - par],
                                acc_sh.at[dst_v.at[0, 0]], ssem).wait()

        @pl.when(jnp.logical_and(j == 0, c + 1 < NCH))
        def _():
          pltpu.async_copy(src_hbm.at[wid, pl.ds((c + 1) * CH, CH)],
                           src_v.at[(c + 1) % 2], isem)
          pltpu.async_copy(dst_hbm.at[wid, pl.ds((c + 1) * CH, CH)],
                           dst_v.at[(c + 1) % 2], isem)

        @pl.when(jnp.logical_and(j == CH - 1, c + 1 < NCH))
        def _():
          pltpu.make_async_copy(src_hbm.at[wid, pl.ds(0, CH)],
                                src_v.at[0], isem).wait()
          pltpu.make_async_copy(dst_hbm.at[wid, pl.ds(0, CH)],
                                dst_v.at[0], isem).wait()

        @pl.when(jj + 1 < NB)
        def _():
          nj = (jj + 1) % CH
          npar = jnp.where(nj == 0, 1 - cpar, cpar)
          pltpu.async_copy(h_hbm.at[src_v.at[npar, nj]],
                           rows_v.at[1 - par], gsem)

        pltpu.async_copy(rows_v.at[par], acc_sh.at[dst_v.at[cpar, j]], ssem,
                         add=True)

    pltpu.make_async_copy(rows_v.at[(NB - 1) % 2], acc_sh.at[dst_v.at[0, 0]],
                          ssem).wait()

    plsc.subcore_barrier()
    pltpu.sync_copy(acc_sh.at[pl.ds(base, STRIPE)],
                    out_hbm.at[cid, pl.ds(base, STRIPE)])

  return agg


_agg128 = _make_agg(D_HID)
_agg64 = _make_agg(D_OUT)

_deg_mesh = plsc.VectorSubcoreMesh(core_axis_name="c", subcore_axis_name="s")


@functools.partial(
    pl.kernel,
    out_type=jax.ShapeDtypeStruct((NC, N_PAD), jnp.float32),
    mesh=_deg_mesh,
    scratch_types=[
        pltpu.VMEM((NB, EB), jnp.int32),
        pltpu.VMEM((EB,), jnp.float32),
        pltpu.VMEM((EB,), jnp.float32),
        pltpu.VMEM_SHARED((N_PAD,), jnp.float32),
        pltpu.SemaphoreType.DMA,
    ],
)
def _deg(dst_hbm, out_hbm, dst_v, ones_v, zero_v, acc_sh, sem):
  del sem
  cid = lax.axis_index("c")
  sid = lax.axis_index("s")
  wid = cid * NS + sid

  pltpu.sync_copy(dst_hbm.at[wid], dst_v)

  z16 = jnp.zeros((16,), jnp.float32)
  o16 = jnp.ones((16,), jnp.float32)
  for k in range(EB // 16):
    zero_v[pl.ds(k * 16, 16)] = z16
    ones_v[pl.ds(k * 16, 16)] = o16

  base = sid * STRIPE
  for kk in range(STRIPE // EB):
    pltpu.sync_copy(zero_v, acc_sh.at[pl.ds(base + kk * EB, EB)])
  plsc.subcore_barrier()

  @pl.loop(0, NB)
  def _(j):
    pltpu.sync_copy(ones_v, acc_sh.at[dst_v.at[j]], add=True)

  plsc.subcore_barrier()
  pltpu.sync_copy(acc_sh.at[pl.ds(base, STRIPE)],
                  out_hbm.at[cid, pl.ds(base, STRIPE)])


def _tc_a(deg_p, x_pad, W1):
  def body(deg_ref, x_ref, w_ref, h_ref):
    dis = lax.rsqrt(deg_ref[0] + deg_ref[1] + 1.0)[:, None]
    h = jnp.dot(x_ref[...], w_ref[...], preferred_element_type=jnp.float32)
    h_ref[...] = h * dis

  return pl.pallas_call(
      body,
      out_shape=jax.ShapeDtypeStruct((N_PAD, D_HID), jnp.float32),
  )(deg_p, x_pad, W1)


def _tc_b(deg_p, agg1, h1, b1, W2):
  def body(deg_ref, a_ref, h_ref, b_ref, w_ref, o_ref):
    dis = lax.rsqrt(deg_ref[0] + deg_ref[1] + 1.0)[:, None]
    t = (a_ref[0] + a_ref[1] + h_ref[...]) * dis + b_ref[...]
    t = jnp.maximum(t, 0.0)
    o_ref[...] = jnp.dot(
        t, w_ref[...], preferred_element_type=jnp.float32) * dis

  return pl.pallas_call(
      body,
      out_shape=jax.ShapeDtypeStruct((N_PAD, D_OUT), jnp.float32),
  )(deg_p, agg1, h1, b1, W2)


def _tc_c(deg_p, agg2, h2, b2):
  def body(deg_ref, a_ref, h_ref, b_ref, o_ref):
    dis = lax.rsqrt(deg_ref[0] + deg_ref[1] + 1.0)[:, None]
    o_ref[...] = (a_ref[0] + a_ref[1] + h_ref[...]) * dis + b_ref[...]

  return pl.pallas_call(
      body,
      out_shape=jax.ShapeDtypeStruct((N_PAD, D_OUT), jnp.float32),
  )(deg_p, agg2, h2, b2)


def kernel(x, edge_index, W1, b1, W2, b2):
  src = edge_index[0]
  dst = edge_index[1]
  pad_e = E_PAD - E
  srcp = jnp.concatenate(
      [src, jnp.zeros((pad_e,), jnp.int32)]).reshape(NW, NB, EB)
  dstp = jnp.concatenate(
      [dst, jnp.full((pad_e,), N, jnp.int32)]).reshape(NW, NB, EB)
  xp = jnp.pad(x, ((0, N_PAD - N), (0, 0)))

  deg_p = _deg(dstp)
  h1 = _tc_a(deg_p, xp, W1)
  agg1 = _agg128(h1, srcp, dstp)
  h2 = _tc_b(deg_p, agg1, h1, b1, W2)
  agg2 = _agg64(h2, srcp, dstp)
  out = _tc_c(deg_p, agg2, h2, b2)
  return out[:N]


# speedup vs baseline: 28.7797x; 2.3468x over previous
import functools

import jax
import jax.numpy as jnp
from jax import lax
from jax.experimental import pallas as pl
from jax.experimental.pallas import tpu as pltpu
from jax.experimental.pallas import tpu_sc as plsc

N = 10000
E = 320000
D_IN = 128
D_HID = 128
D_OUT = 64

NC = 2
NS = 16
NW = NC * NS

N_PAD = 10240
EB = 128
NB = (E // NW + EB - 1) // EB
NB = 80
EPT = NB * EB
E_PAD = NW * EPT
STRIPE = N_PAD // NS


NB16 = 160
EPT16 = NB16 * EB


def _make_agg(D):
  D2 = D // NC
  mesh = plsc.VectorSubcoreMesh(core_axis_name="c", subcore_axis_name="s")

  CH = 16
  NCH = NB16 // CH

  @functools.partial(
      pl.kernel,
      out_type=jax.ShapeDtypeStruct((NC, N_PAD, D2), jnp.float32),
      mesh=mesh,
      scratch_types=[
          pltpu.VMEM((2, CH, EB), jnp.int32),
          pltpu.VMEM((2, CH, EB), jnp.int32),
          pltpu.VMEM((2, EB, D2), jnp.float32),
          pltpu.VMEM_SHARED((N_PAD, D2), jnp.float32),
          pltpu.VMEM_SHARED((N_PAD, D2), jnp.float32),
          pltpu.SemaphoreType.DMA,
          pltpu.SemaphoreType.DMA,
          pltpu.SemaphoreType.DMA,
      ],
      compiler_params=pltpu.CompilerParams(use_tc_tiling_on_sc=False),
  )
  def agg(h_hbm, src_hbm, dst_hbm, out_hbm, src_v, dst_v, rows_v,
          h_sh, acc_sh, gsem, ssem, isem):
    cid = lax.axis_index("c")
    sid = lax.axis_index("s")

    z16 = jnp.zeros((16,), jnp.float32)

    @pl.loop(0, EB)
    def _(r):
      for k in range(D2 // 16):
        rows_v[0, r, pl.ds(k * 16, 16)] = z16

    base = sid * STRIPE
    pltpu.sync_copy(h_hbm.at[cid, pl.ds(base, STRIPE)],
                    h_sh.at[pl.ds(base, STRIPE)])
    for kk in range(STRIPE // EB):
      pltpu.sync_copy(rows_v.at[0], acc_sh.at[pl.ds(base + kk * EB, EB)])
    plsc.subcore_barrier()

    pltpu.sync_copy(src_hbm.at[sid, pl.ds(0, CH)], src_v.at[0])
    pltpu.sync_copy(dst_hbm.at[sid, pl.ds(0, CH)], dst_v.at[0])
    pltpu.async_copy(h_sh.at[src_v.at[0, 0]], rows_v.at[0], gsem)

    @pl.loop(0, NCH)
    def _(c):
      cpar = c % 2

      @pl.loop(0, CH)
      def _(j):
        jj = c * CH + j
        par = jj % 2
        pltpu.make_async_copy(h_sh.at[src_v.at[0, 0]], rows_v.at[par],
                              gsem).wait()
        @pl.when(jj >= 1)
        def _():
          pltpu.make_async_copy(rows_v.at[1 - par],
                                acc_sh.at[dst_v.at[0, 0]], ssem).wait()

        @pl.when(jnp.logical_and(j == 0, c + 1 < NCH))
        def _():
          pltpu.async_copy(src_hbm.at[sid, pl.ds((c + 1) * CH, CH)],
                           src_v.at[(c + 1) % 2], isem)
          pltpu.async_copy(dst_hbm.at[sid, pl.ds((c + 1) * CH, CH)],
                           dst_v.at[(c + 1) % 2], isem)

        @pl.when(jnp.logical_and(j == CH - 1, c + 1 < NCH))
        def _():
          pltpu.make_async_copy(src_hbm.at[sid, pl.ds(0, CH)],
                                src_v.at[0], isem).wait()
          pltpu.make_async_copy(dst_hbm.at[sid, pl.ds(0, CH)],
                                dst_v.at[0], isem).wait()

        @pl.when(jj + 1 < NB16)
        def _():
          nj = (jj + 1) % CH
          npar = jnp.where(nj == 0, 1 - cpar, cpar)
          pltpu.async_copy(h_sh.at[src_v.at[npar, nj]],
                           rows_v.at[1 - par], gsem)

        pltpu.async_copy(rows_v.at[par], acc_sh.at[dst_v.at[cpar, j]], ssem,
                         add=True)

    pltpu.make_async_copy(rows_v.at[(NB16 - 1) % 2], acc_sh.at[dst_v.at[0, 0]],
                          ssem).wait()

    plsc.subcore_barrier()
    pltpu.sync_copy(acc_sh.at[pl.ds(base, STRIPE)],
                    out_hbm.at[cid, pl.ds(base, STRIPE)])

  return agg


_agg128 = _make_agg(D_HID)
_agg64 = _make_agg(D_OUT)

_deg_mesh = plsc.VectorSubcoreMesh(core_axis_name="c", subcore_axis_name="s")


@functools.partial(
    pl.kernel,
    out_type=jax.ShapeDtypeStruct((NC, N_PAD), jnp.float32),
    mesh=_deg_mesh,
    scratch_types=[
        pltpu.VMEM((NB, EB), jnp.int32),
        pltpu.VMEM((EB,), jnp.float32),
        pltpu.VMEM((EB,), jnp.float32),
        pltpu.VMEM_SHARED((N_PAD,), jnp.float32),
        pltpu.SemaphoreType.DMA,
    ],
)
def _deg(dst_hbm, out_hbm, dst_v, ones_v, zero_v, acc_sh, sem):
  del sem
  cid = lax.axis_index("c")
  sid = lax.axis_index("s")
  wid = cid * NS + sid

  pltpu.sync_copy(dst_hbm.at[wid], dst_v)

  z16 = jnp.zeros((16,), jnp.float32)
  o16 = jnp.ones((16,), jnp.float32)
  for k in range(EB // 16):
    zero_v[pl.ds(k * 16, 16)] = z16
    ones_v[pl.ds(k * 16, 16)] = o16

  base = sid * STRIPE
  for kk in range(STRIPE // EB):
    pltpu.sync_copy(zero_v, acc_sh.at[pl.ds(base + kk * EB, EB)])
  plsc.subcore_barrier()

  @pl.loop(0, NB)
  def _(j):
    pltpu.sync_copy(ones_v, acc_sh.at[dst_v.at[j]], add=True)

  plsc.subcore_barrier()
  pltpu.sync_copy(acc_sh.at[pl.ds(base, STRIPE)],
                  out_hbm.at[cid, pl.ds(base, STRIPE)])


def _tc_a(deg_p, x_pad, W1):
  def body(deg_ref, x_ref, w_ref, o_ref):
    dis = lax.rsqrt(deg_ref[0] + deg_ref[1] + 1.0)[:, None]
    h = jnp.dot(x_ref[...], w_ref[...], preferred_element_type=jnp.float32)
    h = h * dis
    o_ref[0] = h[:, :D_HID // 2]
    o_ref[1] = h[:, D_HID // 2:]

  return pl.pallas_call(
      body,
      out_shape=jax.ShapeDtypeStruct((NC, N_PAD, D_HID // 2), jnp.float32),
  )(deg_p, x_pad, W1)


def _tc_b(deg_p, agg1, h1, b1, W2):
  def body(deg_ref, a_ref, h_ref, b_ref, w_ref, o_ref):
    dis = lax.rsqrt(deg_ref[0] + deg_ref[1] + 1.0)[:, None]
    s = jnp.concatenate([a_ref[0] + h_ref[0], a_ref[1] + h_ref[1]], axis=1)
    t = jnp.maximum(s * dis + b_ref[...], 0.0)
    h2 = jnp.dot(t, w_ref[...], preferred_element_type=jnp.float32) * dis
    o_ref[0] = h2[:, :D_OUT // 2]
    o_ref[1] = h2[:, D_OUT // 2:]

  return pl.pallas_call(
      body,
      out_shape=jax.ShapeDtypeStruct((NC, N_PAD, D_OUT // 2), jnp.float32),
  )(deg_p, agg1, h1, b1, W2)


def _tc_c(deg_p, agg2, h2, b2):
  def body(deg_ref, a_ref, h_ref, b_ref, o_ref):
    dis = lax.rsqrt(deg_ref[0] + deg_ref[1] + 1.0)[:, None]
    s = jnp.concatenate([a_ref[0] + h_ref[0], a_ref[1] + h_ref[1]], axis=1)
    o_ref[...] = s * dis + b_ref[...]

  return pl.pallas_call(
      body,
      out_shape=jax.ShapeDtypeStruct((N_PAD, D_OUT), jnp.float32),
  )(deg_p, agg2, h2, b2)


def kernel(x, edge_index, W1, b1, W2, b2):
  src = edge_index[0]
  dst = edge_index[1]
  pad_e = E_PAD - E
  srcf = jnp.concatenate([src, jnp.zeros((pad_e,), jnp.int32)])
  dstf = jnp.concatenate([dst, jnp.full((pad_e,), N, jnp.int32)])
  dstp32 = dstf.reshape(NW, NB, EB)
  srcp16 = srcf.reshape(NS, NB16, EB)
  dstp16 = dstf.reshape(NS, NB16, EB)
  xp = jnp.pad(x, ((0, N_PAD - N), (0, 0)))

  deg_p = _deg(dstp32)
  h1 = _tc_a(deg_p, xp, W1)
  agg1 = _agg128(h1, srcp16, dstp16)
  h2 = _tc_b(deg_p, agg1, h1, b1, W2)
  agg2 = _agg64(h2, srcp16, dstp16)
  out = _tc_c(deg_p, agg2, h2, b2)
  return out[:N]


# speedup vs baseline: 32.5207x; 1.1300x over previous
import functools

import jax
import jax.numpy as jnp
from jax import lax
from jax.experimental import pallas as pl
from jax.experimental.pallas import tpu as pltpu
from jax.experimental.pallas import tpu_sc as plsc

N = 10000
E = 320000
D_IN = 128
D_HID = 128
D_OUT = 64

NC = 2
NS = 16
NW = NC * NS

N_PAD = 10240
EB = 128
NB = (E // NW + EB - 1) // EB
NB = 80
EPT = NB * EB
E_PAD = NW * EPT
STRIPE = N_PAD // NS


NB16 = 160
EPT16 = NB16 * EB


def _make_agg(D):
  D2 = D // NC
  mesh = plsc.VectorSubcoreMesh(core_axis_name="c", subcore_axis_name="s")

  CH = 16
  NCH = NB16 // CH

  @functools.partial(
      pl.kernel,
      out_type=jax.ShapeDtypeStruct((NC, N_PAD, D2), jnp.float32),
      mesh=mesh,
      scratch_types=[
          pltpu.VMEM((2, CH, EB), jnp.int32),
          pltpu.VMEM((2, CH, EB), jnp.int32),
          pltpu.VMEM((4, EB, D2), jnp.float32),
          pltpu.VMEM_SHARED((N_PAD, D2), jnp.float32),
          pltpu.VMEM_SHARED((N_PAD, D2), jnp.float32),
          pltpu.SemaphoreType.DMA,
          pltpu.SemaphoreType.DMA,
          pltpu.SemaphoreType.DMA,
      ],
      compiler_params=pltpu.CompilerParams(use_tc_tiling_on_sc=False),
  )
  def agg(h_hbm, src_hbm, dst_hbm, out_hbm, src_v, dst_v, rows_v,
          h_sh, acc_sh, gsem, ssem, isem):
    cid = lax.axis_index("c")
    sid = lax.axis_index("s")

    z16 = jnp.zeros((16,), jnp.float32)

    @pl.loop(0, EB)
    def _(r):
      for k in range(D2 // 16):
        rows_v[0, r, pl.ds(k * 16, 16)] = z16

    base = sid * STRIPE
    pltpu.sync_copy(h_hbm.at[cid, pl.ds(base, STRIPE)],
                    h_sh.at[pl.ds(base, STRIPE)])
    for kk in range(STRIPE // EB):
      pltpu.sync_copy(rows_v.at[0], acc_sh.at[pl.ds(base + kk * EB, EB)])
    plsc.subcore_barrier()

    pltpu.sync_copy(src_hbm.at[sid, pl.ds(0, CH)], src_v.at[0])
    pltpu.sync_copy(dst_hbm.at[sid, pl.ds(0, CH)], dst_v.at[0])
    pltpu.async_copy(h_sh.at[src_v.at[0, 0]], rows_v.at[0], gsem)
    pltpu.async_copy(h_sh.at[src_v.at[0, 1]], rows_v.at[1], gsem)

    @pl.loop(0, NCH)
    def _(c):
      cpar = c % 2

      @pl.loop(0, CH)
      def _(j):
        jj = c * CH + j
        par = jj % 4
        pltpu.make_async_copy(h_sh.at[src_v.at[0, 0]], rows_v.at[par],
                              gsem).wait()
        @pl.when(jj >= 2)
        def _():
          pltpu.make_async_copy(rows_v.at[(jj + 2) % 4],
                                acc_sh.at[dst_v.at[0, 0]], ssem).wait()

        @pl.when(jnp.logical_and(j == 2, c + 1 < NCH))
        def _():
          pltpu.async_copy(src_hbm.at[sid, pl.ds((c + 1) * CH, CH)],
                           src_v.at[(c + 1) % 2], isem)
          pltpu.async_copy(dst_hbm.at[sid, pl.ds((c + 1) * CH, CH)],
                           dst_v.at[(c + 1) % 2], isem)

        @pl.when(jnp.logical_and(j == CH - 2, c + 1 < NCH))
        def _():
          pltpu.make_async_copy(src_hbm.at[sid, pl.ds(0, CH)],
                                src_v.at[0], isem).wait()
          pltpu.make_async_copy(dst_hbm.at[sid, pl.ds(0, CH)],
                                dst_v.at[0], isem).wait()

        @pl.when(jj + 2 < NB16)
        def _():
          nj = (jj + 2) % CH
          npar = jnp.where(j >= CH - 2, 1 - cpar, cpar)
          pltpu.async_copy(h_sh.at[src_v.at[npar, nj]],
                           rows_v.at[(jj + 2) % 4], gsem)

        pltpu.async_copy(rows_v.at[par], acc_sh.at[dst_v.at[cpar, j]], ssem,
                         add=True)

    pltpu.make_async_copy(rows_v.at[(NB16 - 2) % 4], acc_sh.at[dst_v.at[0, 0]],
                          ssem).wait()
    pltpu.make_async_copy(rows_v.at[(NB16 - 1) % 4], acc_sh.at[dst_v.at[0, 0]],
                          ssem).wait()

    plsc.subcore_barrier()
    pltpu.sync_copy(acc_sh.at[pl.ds(base, STRIPE)],
                    out_hbm.at[cid, pl.ds(base, STRIPE)])

  return agg


_agg128 = _make_agg(D_HID)
_agg64 = _make_agg(D_OUT)

_deg_mesh = plsc.VectorSubcoreMesh(core_axis_name="c", subcore_axis_name="s")


@functools.partial(
    pl.kernel,
    out_type=jax.ShapeDtypeStruct((NC, N_PAD), jnp.float32),
    mesh=_deg_mesh,
    scratch_types=[
        pltpu.VMEM((NB, EB), jnp.int32),
        pltpu.VMEM((EB,), jnp.float32),
        pltpu.VMEM((EB,), jnp.float32),
        pltpu.VMEM_SHARED((N_PAD,), jnp.float32),
        pltpu.SemaphoreType.DMA,
    ],
)
def _deg(dst_hbm, out_hbm, dst_v, ones_v, zero_v, acc_sh, sem):
  del sem
  cid = lax.axis_index("c")
  sid = lax.axis_index("s")
  wid = cid * NS + sid

  pltpu.sync_copy(dst_hbm.at[wid], dst_v)

  z16 = jnp.zeros((16,), jnp.float32)
  o16 = jnp.ones((16,), jnp.float32)
  for k in range(EB // 16):
    zero_v[pl.ds(k * 16, 16)] = z16
    ones_v[pl.ds(k * 16, 16)] = o16

  base = sid * STRIPE
  for kk in range(STRIPE // EB):
    pltpu.sync_copy(zero_v, acc_sh.at[pl.ds(base + kk * EB, EB)])
  plsc.subcore_barrier()

  @pl.loop(0, NB)
  def _(j):
    pltpu.sync_copy(ones_v, acc_sh.at[dst_v.at[j]], add=True)

  plsc.subcore_barrier()
  pltpu.sync_copy(acc_sh.at[pl.ds(base, STRIPE)],
                  out_hbm.at[cid, pl.ds(base, STRIPE)])


def _tc_a(deg_p, x_pad, W1):
  def body(deg_ref, x_ref, w_ref, o_ref):
    dis = lax.rsqrt(deg_ref[0] + deg_ref[1] + 1.0)[:, None]
    h = jnp.dot(x_ref[...], w_ref[...], preferred_element_type=jnp.float32)
    h = h * dis
    o_ref[0] = h[:, :D_HID // 2]
    o_ref[1] = h[:, D_HID // 2:]

  return pl.pallas_call(
      body,
      out_shape=jax.ShapeDtypeStruct((NC, N_PAD, D_HID // 2), jnp.float32),
  )(deg_p, x_pad, W1)


def _tc_b(deg_p, agg1, h1, b1, W2):
  def body(deg_ref, a_ref, h_ref, b_ref, w_ref, o_ref):
    dis = lax.rsqrt(deg_ref[0] + deg_ref[1] + 1.0)[:, None]
    s = jnp.concatenate([a_ref[0] + h_ref[0], a_ref[1] + h_ref[1]], axis=1)
    t = jnp.maximum(s * dis + b_ref[...], 0.0)
    h2 = jnp.dot(t, w_ref[...], preferred_element_type=jnp.float32) * dis
    o_ref[0] = h2[:, :D_OUT // 2]
    o_ref[1] = h2[:, D_OUT // 2:]

  return pl.pallas_call(
      body,
      out_shape=jax.ShapeDtypeStruct((NC, N_PAD, D_OUT // 2), jnp.float32),
  )(deg_p, agg1, h1, b1, W2)


def _tc_c(deg_p, agg2, h2, b2):
  def body(deg_ref, a_ref, h_ref, b_ref, o_ref):
    dis = lax.rsqrt(deg_ref[0] + deg_ref[1] + 1.0)[:, None]
    s = jnp.concatenate([a_ref[0] + h_ref[0], a_ref[1] + h_ref[1]], axis=1)
    o_ref[...] = s * dis + b_ref[...]

  return pl.pallas_call(
      body,
      out_shape=jax.ShapeDtypeStruct((N_PAD, D_OUT), jnp.float32),
  )(deg_p, agg2, h2, b2)


def kernel(x, edge_index, W1, b1, W2, b2):
  src = edge_index[0]
  dst = edge_index[1]
  pad_e = E_PAD - E
  srcf = jnp.concatenate([src, jnp.zeros((pad_e,), jnp.int32)])
  dstf = jnp.concatenate([dst, jnp.full((pad_e,), N, jnp.int32)])
  dstp32 = dstf.reshape(NW, NB, EB)
  srcp16 = srcf.reshape(NS, NB16, EB)
  dstp16 = dstf.reshape(NS, NB16, EB)
  xp = jnp.pad(x, ((0, N_PAD - N), (0, 0)))

  deg_p = _deg(dstp32)
  h1 = _tc_a(deg_p, xp, W1)
  agg1 = _agg128(h1, srcp16, dstp16)
  h2 = _tc_b(deg_p, agg1, h1, b1, W2)
  agg2 = _agg64(h2, srcp16, dstp16)
  out = _tc_c(deg_p, agg2, h2, b2)
  return out[:N]


# speedup vs baseline: 39.8043x; 1.2240x over previous
import functools

import jax
import jax.numpy as jnp
from jax import lax
from jax.experimental import pallas as pl
from jax.experimental.pallas import tpu as pltpu
from jax.experimental.pallas import tpu_sc as plsc

N = 10000
E = 320000
D_IN = 128
D_HID = 128
D_OUT = 64

NC = 2
NS = 16
NW = NC * NS

N_PAD = 10240
EB = 128
NROW = E // EB
STRIPE = N_PAD // NS

NBT = NROW // NS
NTAIL = NROW - NBT * NS
CH = 12
NCH = NBT // CH

NBT32 = NROW // NW
NTAIL32 = NROW - NBT32 * NW


def _make_agg(D):
  D2 = D // NC
  mesh = plsc.VectorSubcoreMesh(core_axis_name="c", subcore_axis_name="s")

  @functools.partial(
      pl.kernel,
      out_type=jax.ShapeDtypeStruct((N_PAD, D), jnp.float32),
      mesh=mesh,
      scratch_types=[
          pltpu.VMEM((2, CH, EB), jnp.int32),
          pltpu.VMEM((2, CH, EB), jnp.int32),
          pltpu.VMEM((4, EB, D2), jnp.float32),
          pltpu.VMEM_SHARED((N_PAD, D2), jnp.float32),
          pltpu.VMEM_SHARED((N_PAD, D2), jnp.float32),
          pltpu.SemaphoreType.DMA,
          pltpu.SemaphoreType.DMA,
          pltpu.SemaphoreType.DMA,
      ],
      compiler_params=pltpu.CompilerParams(use_tc_tiling_on_sc=False),
  )
  def agg(h_hbm, ei_hbm, out_hbm, src_v, dst_v, rows_v,
          h_sh, acc_sh, gsem, ssem, isem):
    cid = lax.axis_index("c")
    sid = lax.axis_index("s")
    row0 = sid * NBT

    z16 = jnp.zeros((16,), jnp.float32)

    @pl.loop(0, EB)
    def _(r):
      for k in range(D2 // 16):
        rows_v[0, r, pl.ds(k * 16, 16)] = z16

    base = sid * STRIPE
    pltpu.sync_copy(h_hbm.at[pl.ds(base, STRIPE), pl.ds(cid * D2, D2)],
                    h_sh.at[pl.ds(base, STRIPE)])
    for kk in range(STRIPE // EB):
      pltpu.sync_copy(rows_v.at[0], acc_sh.at[pl.ds(base + kk * EB, EB)])
    plsc.subcore_barrier()

    pltpu.sync_copy(ei_hbm.at[0, pl.ds(row0, CH)], src_v.at[0])
    pltpu.sync_copy(ei_hbm.at[1, pl.ds(row0, CH)], dst_v.at[0])
    pltpu.async_copy(h_sh.at[src_v.at[0, 0]], rows_v.at[0], gsem)
    pltpu.async_copy(h_sh.at[src_v.at[0, 1]], rows_v.at[1], gsem)

    @pl.loop(0, NCH)
    def _(c):
      cpar = c % 2

      @pl.loop(0, CH)
      def _(j):
        jj = c * CH + j
        par = jj % 4
        pltpu.make_async_copy(h_sh.at[src_v.at[0, 0]], rows_v.at[par],
                              gsem).wait()
        @pl.when(jj >= 2)
        def _():
          pltpu.make_async_copy(rows_v.at[(jj + 2) % 4],
                                acc_sh.at[dst_v.at[0, 0]], ssem).wait()

        @pl.when(jnp.logical_and(j == 2, c + 1 < NCH))
        def _():
          pltpu.async_copy(ei_hbm.at[0, pl.ds(row0 + (c + 1) * CH, CH)],
                           src_v.at[(c + 1) % 2], isem)
          pltpu.async_copy(ei_hbm.at[1, pl.ds(row0 + (c + 1) * CH, CH)],
                           dst_v.at[(c + 1) % 2], isem)

        @pl.when(jnp.logical_and(j == CH - 2, c + 1 < NCH))
        def _():
          pltpu.make_async_copy(ei_hbm.at[0, pl.ds(0, CH)],
                                src_v.at[0], isem).wait()
          pltpu.make_async_copy(ei_hbm.at[1, pl.ds(0, CH)],
                                dst_v.at[0], isem).wait()

        @pl.when(jj + 2 < NBT)
        def _():
          nj = (jj + 2) % CH
          npar = jnp.where(j >= CH - 2, 1 - cpar, cpar)
          pltpu.async_copy(h_sh.at[src_v.at[npar, nj]],
                           rows_v.at[(jj + 2) % 4], gsem)

        pltpu.async_copy(rows_v.at[par], acc_sh.at[dst_v.at[cpar, j]], ssem,
                         add=True)

    pltpu.make_async_copy(rows_v.at[(NBT - 2) % 4], acc_sh.at[dst_v.at[0, 0]],
                          ssem).wait()
    pltpu.make_async_copy(rows_v.at[(NBT - 1) % 4], acc_sh.at[dst_v.at[0, 0]],
                          ssem).wait()

    @pl.when(sid < NTAIL)
    def _():
      pltpu.sync_copy(ei_hbm.at[0, pl.ds(NS * NBT + sid, 1)],
                      src_v.at[0, pl.ds(0, 1)])
      pltpu.sync_copy(ei_hbm.at[1, pl.ds(NS * NBT + sid, 1)],
                      dst_v.at[0, pl.ds(0, 1)])
      pltpu.async_copy(h_sh.at[src_v.at[0, 0]], rows_v.at[0], gsem).wait()
      pltpu.async_copy(rows_v.at[0], acc_sh.at[dst_v.at[0, 0]], ssem,
                       add=True).wait()

    plsc.subcore_barrier()
    pltpu.sync_copy(acc_sh.at[pl.ds(base, STRIPE)],
                    out_hbm.at[pl.ds(base, STRIPE), pl.ds(cid * D2, D2)])

  return agg


_agg128 = _make_agg(D_HID)
_agg64 = _make_agg(D_OUT)

_deg_mesh = plsc.VectorSubcoreMesh(core_axis_name="c", subcore_axis_name="s")


@functools.partial(
    pl.kernel,
    out_type=jax.ShapeDtypeStruct((NC, N_PAD), jnp.float32),
    mesh=_deg_mesh,
    scratch_types=[
        pltpu.VMEM((NBT32, EB), jnp.int32),
        pltpu.VMEM((EB,), jnp.float32),
        pltpu.VMEM((EB,), jnp.float32),
        pltpu.VMEM_SHARED((N_PAD,), jnp.float32),
        pltpu.SemaphoreType.DMA,
    ],
    compiler_params=pltpu.CompilerParams(use_tc_tiling_on_sc=False),
)
def _deg(ei_hbm, out_hbm, dst_v, ones_v, zero_v, acc_sh, sem):
  del sem
  cid = lax.axis_index("c")
  sid = lax.axis_index("s")
  wid = cid * NS + sid

  pltpu.sync_copy(ei_hbm.at[1, pl.ds(wid * NBT32, NBT32)], dst_v)

  z16 = jnp.zeros((16,), jnp.float32)
  o16 = jnp.ones((16,), jnp.float32)
  for k in range(EB // 16):
    zero_v[pl.ds(k * 16, 16)] = z16
    ones_v[pl.ds(k * 16, 16)] = o16

  base = sid * STRIPE
  for kk in range(STRIPE // EB):
    pltpu.sync_copy(zero_v, acc_sh.at[pl.ds(base + kk * EB, EB)])
  plsc.subcore_barrier()

  @pl.loop(0, NBT32)
  def _(j):
    pltpu.sync_copy(ones_v, acc_sh.at[dst_v.at[j]], add=True)

  @pl.when(wid < NTAIL32)
  def _():
    pltpu.sync_copy(ei_hbm.at[1, pl.ds(NW * NBT32 + wid, 1)],
                    dst_v.at[pl.ds(0, 1)])
    pltpu.sync_copy(ones_v, acc_sh.at[dst_v.at[0]], add=True)

  plsc.subcore_barrier()
  pltpu.sync_copy(acc_sh.at[pl.ds(base, STRIPE)],
                  out_hbm.at[cid, pl.ds(base, STRIPE)])


def _tc_a(deg_p, x, W1):
  def body(deg_ref, x_ref, w_ref, o_ref):
    dis = lax.rsqrt(deg_ref[0] + deg_ref[1] + 1.0)[:N, None]
    h = jnp.dot(x_ref[...], w_ref[...], preferred_element_type=jnp.float32)
    o_ref[pl.ds(0, N), :] = h * dis

  return pl.pallas_call(
      body,
      out_shape=jax.ShapeDtypeStruct((N_PAD, D_HID), jnp.float32),
  )(deg_p, x, W1)


def _tc_b(deg_p, agg1, h1, b1, W2):
  def body(deg_ref, a_ref, h_ref, b_ref, w_ref, o_ref):
    dis = lax.rsqrt(deg_ref[0] + deg_ref[1] + 1.0)[:, None]
    t = jnp.maximum((a_ref[...] + h_ref[...]) * dis + b_ref[...], 0.0)
    o_ref[...] = jnp.dot(
        t, w_ref[...], preferred_element_type=jnp.float32) * dis

  return pl.pallas_call(
      body,
      out_shape=jax.ShapeDtypeStruct((N_PAD, D_OUT), jnp.float32),
  )(deg_p, agg1, h1, b1, W2)


def _tc_c(deg_p, agg2, h2, b2):
  def body(deg_ref, a_ref, h_ref, b_ref, o_ref):
    dis = lax.rsqrt(deg_ref[0] + deg_ref[1] + 1.0)[:N, None]
    o_ref[...] = (a_ref[pl.ds(0, N), :] + h_ref[pl.ds(0, N), :]) * dis \
        + b_ref[...]

  return pl.pallas_call(
      body,
      out_shape=jax.ShapeDtypeStruct((N, D_OUT), jnp.float32),
  )(deg_p, agg2, h2, b2)


def kernel(x, edge_index, W1, b1, W2, b2):
  ei3 = edge_index.reshape(2, NROW, EB)

  deg_p = _deg(ei3)
  h1 = _tc_a(deg_p, x, W1)
  agg1 = _agg128(h1, ei3)
  h2 = _tc_b(deg_p, agg1, h1, b1, W2)
  agg2 = _agg64(h2, ei3)
  return _tc_c(deg_p, agg2, h2, b2)


# speedup vs baseline: 41.2087x; 1.0353x over previous
import functools

import jax
import jax.numpy as jnp
from jax import lax
from jax.experimental import pallas as pl
from jax.experimental.pallas import tpu as pltpu
from jax.experimental.pallas import tpu_sc as plsc

N = 10000
E = 320000
D_IN = 128
D_HID = 128
D_OUT = 64

NC = 2
NS = 16
NW = NC * NS

N_PAD = 10240
EB = 128
NROW = E // EB
STRIPE = N_PAD // NS

NBT = NROW // NS
NTAIL = NROW - NBT * NS
CH = 12
NCH = NBT // CH

NBT32 = NROW // NW
NTAIL32 = NROW - NBT32 * NW


def _make_agg(D, edge_split):
  D2 = D // NC
  if edge_split:
    nbt, ntail = NBT32, NTAIL32
    ch = 13
  else:
    nbt, ntail = NBT, NTAIL
    ch = 12
  nch = nbt // ch
  assert ch * nch == nbt and ch >= 5
  mesh = plsc.VectorSubcoreMesh(core_axis_name="c", subcore_axis_name="s")

  @functools.partial(
      pl.kernel,
      out_type=jax.ShapeDtypeStruct((N_PAD, D), jnp.float32),
      mesh=mesh,
      scratch_types=[
          pltpu.VMEM((2, ch, EB), jnp.int32),
          pltpu.VMEM((2, ch, EB), jnp.int32),
          pltpu.VMEM((4, EB, D2), jnp.float32),
          pltpu.VMEM_SHARED((N_PAD, D2), jnp.float32),
          pltpu.VMEM_SHARED((N_PAD, D2), jnp.float32),
          pltpu.SemaphoreType.DMA,
          pltpu.SemaphoreType.DMA,
          pltpu.SemaphoreType.DMA,
      ],
      compiler_params=pltpu.CompilerParams(use_tc_tiling_on_sc=False),
  )
  def agg(h_hbm, ei_hbm, out_hbm, src_v, dst_v, rows_v,
          h_sh, acc_sh, gsem, ssem, isem):
    cid = lax.axis_index("c")
    sid = lax.axis_index("s")
    if edge_split:
      worker = cid * NS + sid
    else:
      worker = sid
    row0 = worker * nbt

    z16 = jnp.zeros((16,), jnp.float32)

    @pl.loop(0, EB)
    def _(r):
      for k in range(D2 // 16):
        rows_v[0, r, pl.ds(k * 16, 16)] = z16

    base = sid * STRIPE
    pltpu.sync_copy(h_hbm.at[pl.ds(base, STRIPE), pl.ds(cid * D2, D2)],
                    h_sh.at[pl.ds(base, STRIPE)])
    for kk in range(STRIPE // EB):
      pltpu.sync_copy(rows_v.at[0], acc_sh.at[pl.ds(base + kk * EB, EB)])
    plsc.subcore_barrier()

    pltpu.sync_copy(ei_hbm.at[0, pl.ds(row0, ch)], src_v.at[0])
    pltpu.sync_copy(ei_hbm.at[1, pl.ds(row0, ch)], dst_v.at[0])
    pltpu.async_copy(h_sh.at[src_v.at[0, 0]], rows_v.at[0], gsem)
    pltpu.async_copy(h_sh.at[src_v.at[0, 1]], rows_v.at[1], gsem)

    @pl.loop(0, nch)
    def _(c):
      cpar = c % 2

      @pl.loop(0, ch)
      def _(j):
        jj = c * ch + j
        par = jj % 4
        pltpu.make_async_copy(h_sh.at[src_v.at[0, 0]], rows_v.at[par],
                              gsem).wait()
        @pl.when(jj >= 2)
        def _():
          pltpu.make_async_copy(rows_v.at[(jj + 2) % 4],
                                acc_sh.at[dst_v.at[0, 0]], ssem).wait()

        @pl.when(jnp.logical_and(j == 2, c + 1 < nch))
        def _():
          pltpu.async_copy(ei_hbm.at[0, pl.ds(row0 + (c + 1) * ch, ch)],
                           src_v.at[(c + 1) % 2], isem)
          pltpu.async_copy(ei_hbm.at[1, pl.ds(row0 + (c + 1) * ch, ch)],
                           dst_v.at[(c + 1) % 2], isem)

        @pl.when(jnp.logical_and(j == ch - 2, c + 1 < nch))
        def _():
          pltpu.make_async_copy(ei_hbm.at[0, pl.ds(0, ch)],
                                src_v.at[0], isem).wait()
          pltpu.make_async_copy(ei_hbm.at[1, pl.ds(0, ch)],
                                dst_v.at[0], isem).wait()

        @pl.when(jj + 2 < nbt)
        def _():
          nj = (jj + 2) % ch
          npar = jnp.where(j >= ch - 2, 1 - cpar, cpar)
          pltpu.async_copy(h_sh.at[src_v.at[npar, nj]],
                           rows_v.at[(jj + 2) % 4], gsem)

        pltpu.async_copy(rows_v.at[par], acc_sh.at[dst_v.at[cpar, j]], ssem,
                         add=True)

    pltpu.make_async_copy(rows_v.at[(nbt - 2) % 4], acc_sh.at[dst_v.at[0, 0]],
                          ssem).wait()
    pltpu.make_async_copy(rows_v.at[(nbt - 1) % 4], acc_sh.at[dst_v.at[0, 0]],
                          ssem).wait()

    @pl.when(worker < ntail)
    def _():
      pltpu.sync_copy(ei_hbm.at[0, pl.ds(NROW - ntail + worker, 1)],
                      src_v.at[0, pl.ds(0, 1)])
      pltpu.sync_copy(ei_hbm.at[1, pl.ds(NROW - ntail + worker, 1)],
                      dst_v.at[0, pl.ds(0, 1)])
      pltpu.async_copy(h_sh.at[src_v.at[0, 0]], rows_v.at[0], gsem).wait()
      pltpu.async_copy(rows_v.at[0], acc_sh.at[dst_v.at[0, 0]], ssem,
                       add=True).wait()

    plsc.subcore_barrier()
    pltpu.sync_copy(acc_sh.at[pl.ds(base, STRIPE)],
                    out_hbm.at[pl.ds(base, STRIPE), pl.ds(cid * D2, D2)])

  return agg


_agg128 = _make_agg(D_HID, edge_split=False)
_agg64 = _make_agg(2 * D_OUT, edge_split=True)

_deg_mesh = plsc.VectorSubcoreMesh(core_axis_name="c", subcore_axis_name="s")


@functools.partial(
    pl.kernel,
    out_type=jax.ShapeDtypeStruct((NC, N_PAD), jnp.float32),
    mesh=_deg_mesh,
    scratch_types=[
        pltpu.VMEM((NBT32, EB), jnp.int32),
        pltpu.VMEM((EB,), jnp.float32),
        pltpu.VMEM((EB,), jnp.float32),
        pltpu.VMEM_SHARED((N_PAD,), jnp.float32),
        pltpu.SemaphoreType.DMA,
    ],
    compiler_params=pltpu.CompilerParams(use_tc_tiling_on_sc=False),
)
def _deg(ei_hbm, out_hbm, dst_v, ones_v, zero_v, acc_sh, sem):
  del sem
  cid = lax.axis_index("c")
  sid = lax.axis_index("s")
  wid = cid * NS + sid

  pltpu.sync_copy(ei_hbm.at[1, pl.ds(wid * NBT32, NBT32)], dst_v)

  z16 = jnp.zeros((16,), jnp.float32)
  o16 = jnp.ones((16,), jnp.float32)
  for k in range(EB // 16):
    zero_v[pl.ds(k * 16, 16)] = z16
    ones_v[pl.ds(k * 16, 16)] = o16

  base = sid * STRIPE
  for kk in range(STRIPE // EB):
    pltpu.sync_copy(zero_v, acc_sh.at[pl.ds(base + kk * EB, EB)])
  plsc.subcore_barrier()

  @pl.loop(0, NBT32)
  def _(j):
    pltpu.sync_copy(ones_v, acc_sh.at[dst_v.at[j]], add=True)

  @pl.when(wid < NTAIL32)
  def _():
    pltpu.sync_copy(ei_hbm.at[1, pl.ds(NW * NBT32 + wid, 1)],
                    dst_v.at[pl.ds(0, 1)])
    pltpu.sync_copy(ones_v, acc_sh.at[dst_v.at[0]], add=True)

  plsc.subcore_barrier()
  pltpu.sync_copy(acc_sh.at[pl.ds(base, STRIPE)],
                  out_hbm.at[cid, pl.ds(base, STRIPE)])


def _tc_a(deg_p, x, W1):
  def body(deg_ref, x_ref, w_ref, o_ref):
    dis = lax.rsqrt(deg_ref[0] + deg_ref[1] + 1.0)[:N, None]
    h = jnp.dot(x_ref[...], w_ref[...], preferred_element_type=jnp.float32)
    o_ref[pl.ds(0, N), :] = h * dis

  return pl.pallas_call(
      body,
      out_shape=jax.ShapeDtypeStruct((N_PAD, D_HID), jnp.float32),
  )(deg_p, x, W1)


def _tc_b(deg_p, agg1, h1, b1, W2):
  def body(deg_ref, a_ref, h_ref, b_ref, w_ref, o_ref):
    dis = lax.rsqrt(deg_ref[0] + deg_ref[1] + 1.0)[:, None]
    t = jnp.maximum((a_ref[...] + h_ref[...]) * dis + b_ref[...], 0.0)
    h2 = jnp.dot(t, w_ref[...], preferred_element_type=jnp.float32) * dis
    o_ref[:, :D_OUT] = h2
    o_ref[:, D_OUT:] = h2

  return pl.pallas_call(
      body,
      out_shape=jax.ShapeDtypeStruct((N_PAD, 2 * D_OUT), jnp.float32),
  )(deg_p, agg1, h1, b1, W2)


def _tc_c(deg_p, agg2, h2, b2):
  def body(deg_ref, a_ref, h_ref, b_ref, o_ref):
    dis = lax.rsqrt(deg_ref[0] + deg_ref[1] + 1.0)[:N, None]
    s = (a_ref[pl.ds(0, N), pl.ds(0, D_OUT)]
         + a_ref[pl.ds(0, N), pl.ds(D_OUT, D_OUT)]
         + h_ref[pl.ds(0, N), pl.ds(0, D_OUT)])
    o_ref[...] = s * dis + b_ref[...]

  return pl.pallas_call(
      body,
      out_shape=jax.ShapeDtypeStruct((N, D_OUT), jnp.float32),
  )(deg_p, agg2, h2, b2)


def kernel(x, edge_index, W1, b1, W2, b2):
  ei3 = edge_index.reshape(2, NROW, EB)

  deg_p = _deg(ei3)
  h1 = _tc_a(deg_p, x, W1)
  agg1 = _agg128(h1, ei3)
  h2 = _tc_b(deg_p, agg1, h1, b1, W2)
  agg2 = _agg64(h2, ei3)
  return _tc_c(deg_p, agg2, h2, b2)
